# baseline (speedup 1.0000x reference)
import functools

import jax
import jax.numpy as jnp
from jax import lax
from jax.experimental import pallas as pl
from jax.experimental.pallas import tpu as pltpu
from jax.experimental.pallas import tpu_sc as plsc

_B = 1024
_D = 64
_MARGIN = 1.0
_N_PAIRS = _B * (_B - 1) // 2

_L = 16
_NW = 16
_ROWS = _B // _NW
_NCH = 5
_DP = _NCH * _L
_CP = 128


def _sc_seg_kernel(emb_hbm, tgt_hbm, out_hbm, rows_v, tgt_v, accl_v, sem):
    wid = lax.axis_index("s")
    base = wid * _ROWS

    pltpu.sync_copy(emb_hbm.at[pl.ds(base, _ROWS)], rows_v)
    pltpu.sync_copy(tgt_hbm.at[pl.ds(base, _ROWS)], tgt_v)

    zvec = jnp.zeros((_L,), jnp.float32)

    def zero_row(i, _):
        for j in range(_NCH):
            accl_v[i, pl.ds(j * _L, _L)] = zvec
        return 0

    lax.fori_loop(0, _CP, zero_row, 0)

    lane = lax.iota(jnp.int32, _L)
    dn = lax.GatherDimensionNumbers(
        offset_dims=(), collapsed_slice_dims=(0,), start_index_map=(0,))
    for blk in range(_ROWS // _L):
        idx16 = tgt_v[pl.ds(blk * _L, _L)]
        for rr in range(_L):
            r = blk * _L + rr
            cls_vec = lax.gather(
                idx16, jnp.full((_L, 1), rr, jnp.int32), dn, (1,),
                mode=lax.GatherScatterMode.PROMISE_IN_BOUNDS)
            c0 = rows_v[r, pl.ds(0, _L)]
            c1 = rows_v[r, pl.ds(_L, _L)]
            c2 = rows_v[r, pl.ds(2 * _L, _L)]
            c3 = rows_v[r, pl.ds(3 * _L, _L)]
            sumsq = jnp.sum(c0 * c0 + c1 * c1 + c2 * c2 + c3 * c3)
            c4 = jnp.where(lane == 0, sumsq,
                           jnp.where(lane == 1, jnp.float32(1.0),
                                     jnp.float32(0.0)))
            for j, cj in enumerate((c0, c1, c2, c3, c4)):
                plsc.addupdate_scatter(
                    accl_v, [cls_vec, jnp.int32(j * _L) + lane], cj)

    for k in range(_NW):
        @pl.when(wid == k)
        def _():
            pltpu.sync_copy(accl_v, out_hbm.at[k])


_sc_seg = functools.partial(
    pl.kernel,
    out_type=jax.ShapeDtypeStruct((_NW, _CP, _DP), jnp.float32),
    mesh=plsc.VectorSubcoreMesh(
        core_axis_name="c", subcore_axis_name="s", num_cores=1),
    scratch_types=[
        pltpu.VMEM((_ROWS, _D), jnp.float32),
        pltpu.VMEM((_ROWS,), jnp.int32),
        pltpu.VMEM((_CP, _DP), jnp.float32),
        pltpu.SemaphoreType.DMA,
    ],
    compiler_params=pltpu.CompilerParams(needs_layout_passes=False),
)(_sc_seg_kernel)


def _tc_kernel(e_ref, t_ref, a_ref, out_ref):
    e = e_ref[...]
    t = t_ref[...]
    g = lax.dot_general(
        e, e, (((1,), (1,)), ((), ())), preferred_element_type=jnp.float32
    )
    nrm = jnp.sum(e * e, axis=1, keepdims=True)
    d2 = jnp.maximum(nrm + nrm.T - 2.0 * g, 0.0)
    neg = jnp.maximum(_MARGIN - jnp.sqrt(d2 + 1e-6), 0.0)
    diff = t != t.reshape(1, _B)
    neg_total = 0.5 * jnp.sum(jnp.where(diff, neg * neg, 0.0))

    acc = jnp.sum(a_ref[...], axis=0)
    ssum = jnp.sum(acc[:, :_D] * acc[:, :_D], axis=1)
    m = acc[:, _D]
    n = acc[:, _D + 1]
    pos_total = jnp.sum(n * m - ssum)

    out_ref[...] = ((neg_total + pos_total)
                    / jnp.float32(_N_PAIRS)).reshape(1, 1)


def kernel(embeddings, target):
    accs = _sc_seg(embeddings, target)
    out = pl.pallas_call(
        _tc_kernel,
        out_shape=jax.ShapeDtypeStruct((1, 1), jnp.float32),
    )(embeddings, target.reshape(_B, 1), accs)
    return out[0, 0]
